# baseline (speedup 1.0000x reference)
import jax
import jax.numpy as jnp
from jax.experimental import pallas as pl
from jax.experimental.pallas import tpu as pltpu

N_EMB = 1024
DIM = 64
COMMITMENT_COST = 0.25
BLK = 4096
R = 256
CG = 128
NG = N_EMB // CG


def _vq_block(x_ref, c_ref, q_ref, idx_ref, loss_ref, perp_ref,
              counts_ref, cs_ref, n2c_ref, idxcol_ref):
    i = pl.program_id(0)
    nsteps = pl.num_programs(0)
    total_tokens = nsteps * BLK

    @pl.when(i == 0)
    def _prep():
        c0 = c_ref[...]
        cs_ref[...] = jnp.sum(c0 * c0, axis=1)[None, :]
        n2c_ref[...] = c0 * (-2.0)

    def row_chunk(r, _):
        xc = x_ref[pl.ds(r * R, R), :]
        is_c = jnp.sum(xc * xc, axis=1, keepdims=True)
        m_run = jnp.full((R, 1), jnp.inf, jnp.float32)
        i_run = jnp.zeros((R, 1), jnp.int32)
        for g in range(NG):
            mm2 = jnp.dot(xc, n2c_ref[g * CG:(g + 1) * CG, :].T,
                          preferred_element_type=jnp.float32)
            d = (is_c + cs_ref[0, g * CG:(g + 1) * CG]) + mm2
            lm = jnp.min(d, axis=1, keepdims=True)
            col = jax.lax.broadcasted_iota(jnp.int32, d.shape, 1) + g * CG
            li = jnp.min(jnp.where(d == lm, col, N_EMB), axis=1,
                         keepdims=True)
            better = lm < m_run
            m_run = jnp.where(better, lm, m_run)
            i_run = jnp.where(better, li, i_run)
        idxcol_ref[pl.ds(r * R, R), :] = i_run
        return _

    jax.lax.fori_loop(0, BLK // R, row_chunk, None)

    x = x_ref[...]
    idx_col = idxcol_ref[...]
    col = jax.lax.broadcasted_iota(jnp.int32, (BLK, N_EMB), 1)
    oh = (col == idx_col).astype(jnp.float32)
    q = jnp.dot(oh, c_ref[...], preferred_element_type=jnp.float32)

    q_ref[...] = q
    idx_ref[0, 0, :] = idx_col[:, 0]

    blk_loss = jnp.sum((q - x) ** 2)
    ones_row = jnp.ones((1, BLK), jnp.float32)
    blk_counts = jnp.dot(ones_row, oh, preferred_element_type=jnp.float32)

    @pl.when(i == 0)
    def _init():
        counts_ref[...] = blk_counts
        loss_ref[...] = blk_loss.reshape(1, 1)
        perp_ref[...] = jnp.zeros((1, 1), jnp.float32)

    @pl.when(i > 0)
    def _acc():
        counts_ref[...] += blk_counts
        loss_ref[...] += blk_loss.reshape(1, 1)

    @pl.when(i == nsteps - 1)
    def _finish():
        p = counts_ref[0, :] / jnp.float32(total_tokens)
        ent = jnp.sum(p * jnp.log(p + 1e-10))
        perp_ref[...] = jnp.exp(-ent).reshape(1, 1)
        mse = loss_ref[0, 0] / jnp.float32(total_tokens * DIM)
        loss_ref[...] = (mse * COMMITMENT_COST + mse).reshape(1, 1)


def kernel(inputs, codebook):
    input_shape = inputs.shape
    x = inputs.reshape(-1, DIM)
    tokens = x.shape[0]
    grid = tokens // BLK

    q, idx3, vq_loss, perp = pl.pallas_call(
        _vq_block,
        grid=(grid,),
        in_specs=[
            pl.BlockSpec((BLK, DIM), lambda i: (i, 0)),
            pl.BlockSpec((N_EMB, DIM), lambda i: (0, 0)),
        ],
        out_specs=[
            pl.BlockSpec((BLK, DIM), lambda i: (i, 0)),
            pl.BlockSpec((1, 1, BLK), lambda i: (i, 0, 0)),
            pl.BlockSpec((1, 1), lambda i: (0, 0)),
            pl.BlockSpec((1, 1), lambda i: (0, 0)),
        ],
        out_shape=[
            jax.ShapeDtypeStruct((tokens, DIM), jnp.float32),
            jax.ShapeDtypeStruct((grid, 1, BLK), jnp.int32),
            jax.ShapeDtypeStruct((1, 1), jnp.float32),
            jax.ShapeDtypeStruct((1, 1), jnp.float32),
        ],
        scratch_shapes=[
            pltpu.VMEM((1, N_EMB), jnp.float32),
            pltpu.VMEM((1, N_EMB), jnp.float32),
            pltpu.VMEM((N_EMB, DIM), jnp.float32),
            pltpu.VMEM((BLK, 1), jnp.int32),
        ],
    )(x, codebook)

    quantized = q.reshape(input_shape)
    indices = idx3.reshape(input_shape[:-1])
    return (quantized, vq_loss[0, 0], perp[0, 0], indices)


# speedup vs baseline: 1.6183x; 1.6183x over previous
import jax
import jax.numpy as jnp
from jax.experimental import pallas as pl
from jax.experimental.pallas import tpu as pltpu

N_EMB = 1024
DIM = 64
COMMITMENT_COST = 0.25
BLK = 4096
R = 256
CG = 256
NG = N_EMB // CG


def _vq_block(x_ref, c_ref, q_ref, idx_ref, loss_ref, perp_ref,
              counts_ref, cs_ref, n2c_ref, idxcol_ref):
    i = pl.program_id(0)
    nsteps = pl.num_programs(0)
    total_tokens = nsteps * BLK

    @pl.when(i == 0)
    def _prep():
        c0 = c_ref[...]
        cs_ref[...] = jnp.sum(c0 * c0, axis=1)[None, :]
        n2c_ref[...] = c0 * (-2.0)

    def row_chunk(r, _):
        xc = x_ref[pl.ds(r * R, R), :]
        is_c = jnp.sum(xc * xc, axis=1, keepdims=True)
        m_run = jnp.full((R, 1), jnp.inf, jnp.float32)
        i_run = jnp.zeros((R, 1), jnp.int32)
        for g in range(NG):
            mm2 = jnp.dot(xc, n2c_ref[g * CG:(g + 1) * CG, :].T,
                          preferred_element_type=jnp.float32)
            d = (is_c + cs_ref[0, g * CG:(g + 1) * CG]) + mm2
            lm = jnp.min(d, axis=1, keepdims=True)
            col = jax.lax.broadcasted_iota(jnp.int32, d.shape, 1) + g * CG
            li = jnp.min(jnp.where(d == lm, col, N_EMB), axis=1,
                         keepdims=True)
            better = lm < m_run
            m_run = jnp.where(better, lm, m_run)
            i_run = jnp.where(better, li, i_run)
        idxcol_ref[pl.ds(r * R, R), :] = i_run
        return _

    jax.lax.fori_loop(0, BLK // R, row_chunk, None)

    x = x_ref[...]
    idx_col = idxcol_ref[...]
    col = jax.lax.broadcasted_iota(jnp.int32, (BLK, N_EMB), 1)
    oh = (col == idx_col).astype(jnp.float32)
    q = jnp.dot(oh, c_ref[...], preferred_element_type=jnp.float32)

    q_ref[...] = q
    idx_ref[0, 0, :] = idx_col[:, 0]

    blk_loss = jnp.sum((q - x) ** 2)
    ones_row = jnp.ones((1, BLK), jnp.float32)
    blk_counts = jnp.dot(ones_row, oh, preferred_element_type=jnp.float32)

    @pl.when(i == 0)
    def _init():
        counts_ref[...] = blk_counts
        loss_ref[...] = blk_loss.reshape(1, 1)
        perp_ref[...] = jnp.zeros((1, 1), jnp.float32)

    @pl.when(i > 0)
    def _acc():
        counts_ref[...] += blk_counts
        loss_ref[...] += blk_loss.reshape(1, 1)

    @pl.when(i == nsteps - 1)
    def _finish():
        p = counts_ref[0, :] / jnp.float32(total_tokens)
        ent = jnp.sum(p * jnp.log(p + 1e-10))
        perp_ref[...] = jnp.exp(-ent).reshape(1, 1)
        mse = loss_ref[0, 0] / jnp.float32(total_tokens * DIM)
        loss_ref[...] = (mse * COMMITMENT_COST + mse).reshape(1, 1)


def kernel(inputs, codebook):
    input_shape = inputs.shape
    x = inputs.reshape(-1, DIM)
    tokens = x.shape[0]
    grid = tokens // BLK

    q, idx3, vq_loss, perp = pl.pallas_call(
        _vq_block,
        grid=(grid,),
        in_specs=[
            pl.BlockSpec((BLK, DIM), lambda i: (i, 0)),
            pl.BlockSpec((N_EMB, DIM), lambda i: (0, 0)),
        ],
        out_specs=[
            pl.BlockSpec((BLK, DIM), lambda i: (i, 0)),
            pl.BlockSpec((1, 1, BLK), lambda i: (i, 0, 0)),
            pl.BlockSpec((1, 1), lambda i: (0, 0)),
            pl.BlockSpec((1, 1), lambda i: (0, 0)),
        ],
        out_shape=[
            jax.ShapeDtypeStruct((tokens, DIM), jnp.float32),
            jax.ShapeDtypeStruct((grid, 1, BLK), jnp.int32),
            jax.ShapeDtypeStruct((1, 1), jnp.float32),
            jax.ShapeDtypeStruct((1, 1), jnp.float32),
        ],
        scratch_shapes=[
            pltpu.VMEM((1, N_EMB), jnp.float32),
            pltpu.VMEM((1, N_EMB), jnp.float32),
            pltpu.VMEM((N_EMB, DIM), jnp.float32),
            pltpu.VMEM((BLK, 1), jnp.int32),
        ],
    )(x, codebook)

    quantized = q.reshape(input_shape)
    indices = idx3.reshape(input_shape[:-1])
    return (quantized, vq_loss[0, 0], perp[0, 0], indices)


# speedup vs baseline: 1.7621x; 1.0889x over previous
import functools

import jax
import jax.numpy as jnp
from jax import lax
from jax.experimental import pallas as pl
from jax.experimental.pallas import tpu as pltpu
from jax.experimental.pallas import tpu_sc as plsc

N_EMB = 1024
DIM = 64
COMMITMENT_COST = 0.25
BLK = 4096
TOKENS = 16384
NW = 32
BPW = TOKENS // NW


def _dist_block(x_ref, c_ref, idx_ref, loss_ref, perp_ref,
                cs_ref, n2c_ref, counts_ref):
    i = pl.program_id(0)
    nsteps = pl.num_programs(0)

    @pl.when(i == 0)
    def _prep():
        c0 = c_ref[...]
        cs_ref[...] = jnp.sum(c0 * c0, axis=1)[None, :]
        n2c_ref[...] = c0 * (-2.0)

    x = x_ref[...]
    input_sq = jnp.sum(x * x, axis=1, keepdims=True)
    mm2 = jnp.dot(x, n2c_ref[...].T, preferred_element_type=jnp.float32)
    d = (input_sq + cs_ref[0, :]) + mm2

    dmin = jnp.min(d, axis=1, keepdims=True)
    col = jax.lax.broadcasted_iota(jnp.int32, d.shape, 1)
    idx = jnp.min(jnp.where(d == dmin, col, N_EMB), axis=1)

    idx_ref[0, 0, :] = idx

    blk_loss = jnp.sum(dmin)
    oh = (col == idx[:, None]).astype(jnp.bfloat16)
    ones_row = jnp.ones((1, BLK), jnp.bfloat16)
    blk_counts = jnp.dot(ones_row, oh, preferred_element_type=jnp.float32)

    @pl.when(i == 0)
    def _init():
        counts_ref[...] = blk_counts
        loss_ref[...] = blk_loss.reshape(1, 1)
        perp_ref[...] = jnp.zeros((1, 1), jnp.float32)

    @pl.when(i > 0)
    def _acc():
        counts_ref[...] += blk_counts
        loss_ref[...] += blk_loss.reshape(1, 1)

    @pl.when(i == nsteps - 1)
    def _finish():
        p = counts_ref[0, :] / jnp.float32(TOKENS)
        ent = jnp.sum(p * jnp.log(p + 1e-10))
        perp_ref[...] = jnp.exp(-ent).reshape(1, 1)
        mse = loss_ref[0, 0] / jnp.float32(TOKENS * DIM)
        loss_ref[...] = (mse * COMMITMENT_COST + mse).reshape(1, 1)


def _sc_lookup(table_hbm, idx_hbm, q_hbm, idx_v, rows_v, sem):
    wid = lax.axis_index("s") * 2 + lax.axis_index("c")
    base = wid * BPW
    pltpu.sync_copy(idx_hbm.at[pl.ds(base, BPW)], idx_v)
    pltpu.async_copy(table_hbm.at[idx_v], rows_v, sem).wait()
    pltpu.sync_copy(rows_v, q_hbm.at[pl.ds(base, BPW)])


def kernel(inputs, codebook):
    input_shape = inputs.shape
    x = inputs.reshape(-1, DIM)
    grid = TOKENS // BLK

    idx3, vq_loss, perp = pl.pallas_call(
        _dist_block,
        grid=(grid,),
        in_specs=[
            pl.BlockSpec((BLK, DIM), lambda i: (i, 0)),
            pl.BlockSpec((N_EMB, DIM), lambda i: (0, 0)),
        ],
        out_specs=[
            pl.BlockSpec((1, 1, BLK), lambda i: (i, 0, 0)),
            pl.BlockSpec((1, 1), lambda i: (0, 0)),
            pl.BlockSpec((1, 1), lambda i: (0, 0)),
        ],
        out_shape=[
            jax.ShapeDtypeStruct((grid, 1, BLK), jnp.int32),
            jax.ShapeDtypeStruct((1, 1), jnp.float32),
            jax.ShapeDtypeStruct((1, 1), jnp.float32),
        ],
        scratch_shapes=[
            pltpu.VMEM((1, N_EMB), jnp.float32),
            pltpu.VMEM((N_EMB, DIM), jnp.float32),
            pltpu.VMEM((1, N_EMB), jnp.float32),
        ],
    )(x, codebook)

    idx_flat = idx3.reshape(TOKENS)

    sc_fn = functools.partial(
        pl.kernel,
        mesh=plsc.VectorSubcoreMesh(core_axis_name="c", subcore_axis_name="s"),
        compiler_params=pltpu.CompilerParams(use_tc_tiling_on_sc=False),
        out_type=jax.ShapeDtypeStruct((TOKENS, DIM), jnp.float32),
        scratch_types=[
            pltpu.VMEM((BPW,), jnp.int32),
            pltpu.VMEM((BPW, DIM), jnp.float32),
            pltpu.SemaphoreType.DMA,
        ],
    )(_sc_lookup)
    q = sc_fn(codebook, idx_flat)

    quantized = q.reshape(input_shape)
    indices = idx_flat.reshape(input_shape[:-1])
    return (quantized, vq_loss[0, 0], perp[0, 0], indices)


# speedup vs baseline: 2.1836x; 1.2392x over previous
import jax
import jax.numpy as jnp
from jax.experimental import pallas as pl
from jax.experimental.pallas import tpu as pltpu

N_EMB = 1024
DIM = 64
COMMITMENT_COST = 0.25
BLK = 4096


def _vq_block(x_ref, c_ref, q_ref, idx_ref, loss_ref, perp_ref,
              counts_ref, cs_ref, n2c_ref):
    i = pl.program_id(0)
    nsteps = pl.num_programs(0)
    total_tokens = nsteps * BLK

    @pl.when(i == 0)
    def _prep():
        c0 = c_ref[...]
        cs_ref[...] = jnp.sum(c0 * c0, axis=1)[None, :]
        n2c_ref[...] = c0 * (-2.0)

    x = x_ref[...]

    input_sq = jnp.sum(x * x, axis=1, keepdims=True)
    mm2 = jnp.dot(x, n2c_ref[...].T, preferred_element_type=jnp.float32)
    d = (input_sq + cs_ref[0, :]) + mm2

    dmin = jnp.min(d, axis=1, keepdims=True)
    col = jax.lax.broadcasted_iota(jnp.int32, d.shape, 1)
    idx = jnp.min(jnp.where(d == dmin, col, N_EMB), axis=1)

    oh = (col == idx[:, None]).astype(jnp.bfloat16)
    q = jax.lax.dot_general(oh, c_ref[...], (((1,), (0,)), ((), ())),
                            preferred_element_type=jnp.float32)

    q_ref[...] = q
    idx_ref[0, 0, :] = idx

    blk_loss = jnp.sum((q - x) ** 2)
    ones_row = jnp.ones((1, BLK), jnp.bfloat16)
    blk_counts = jax.lax.dot_general(ones_row, oh, (((1,), (0,)), ((), ())),
                                     preferred_element_type=jnp.float32)

    @pl.when(i == 0)
    def _init():
        counts_ref[...] = blk_counts
        loss_ref[...] = blk_loss.reshape(1, 1)
        perp_ref[...] = jnp.zeros((1, 1), jnp.float32)

    @pl.when(i > 0)
    def _acc():
        counts_ref[...] += blk_counts
        loss_ref[...] += blk_loss.reshape(1, 1)

    @pl.when(i == nsteps - 1)
    def _finish():
        p = counts_ref[0, :] / jnp.float32(total_tokens)
        ent = jnp.sum(p * jnp.log(p + 1e-10))
        perp_ref[...] = jnp.exp(-ent).reshape(1, 1)
        mse = loss_ref[0, 0] / jnp.float32(total_tokens * DIM)
        loss_ref[...] = (mse * COMMITMENT_COST + mse).reshape(1, 1)


def kernel(inputs, codebook):
    input_shape = inputs.shape
    x = inputs.reshape(-1, DIM)
    tokens = x.shape[0]
    grid = tokens // BLK

    q, idx3, vq_loss, perp = pl.pallas_call(
        _vq_block,
        grid=(grid,),
        in_specs=[
            pl.BlockSpec((BLK, DIM), lambda i: (i, 0)),
            pl.BlockSpec((N_EMB, DIM), lambda i: (0, 0)),
        ],
        out_specs=[
            pl.BlockSpec((BLK, DIM), lambda i: (i, 0)),
            pl.BlockSpec((1, 1, BLK), lambda i: (i, 0, 0)),
            pl.BlockSpec((1, 1), lambda i: (0, 0)),
            pl.BlockSpec((1, 1), lambda i: (0, 0)),
        ],
        out_shape=[
            jax.ShapeDtypeStruct((tokens, DIM), jnp.float32),
            jax.ShapeDtypeStruct((grid, 1, BLK), jnp.int32),
            jax.ShapeDtypeStruct((1, 1), jnp.float32),
            jax.ShapeDtypeStruct((1, 1), jnp.float32),
        ],
        scratch_shapes=[
            pltpu.VMEM((1, N_EMB), jnp.float32),
            pltpu.VMEM((1, N_EMB), jnp.float32),
            pltpu.VMEM((N_EMB, DIM), jnp.float32),
        ],
    )(x, codebook)

    quantized = q.reshape(input_shape)
    indices = idx3.reshape(input_shape[:-1])
    return (quantized, vq_loss[0, 0], perp[0, 0], indices)


# speedup vs baseline: 2.2071x; 1.0107x over previous
import jax
import jax.numpy as jnp
from jax.experimental import pallas as pl
from jax.experimental.pallas import tpu as pltpu

N_EMB = 1024
DIM = 64
COMMITMENT_COST = 0.25
BLK = 4096


def _vq_block(x_ref, c_ref, q_ref, idx_ref, loss_ref, perp_ref,
              counts_ref, cs_ref, n2c_ref):
    i = pl.program_id(0)
    nsteps = pl.num_programs(0)
    total_tokens = nsteps * BLK

    @pl.when(i == 0)
    def _prep():
        c0 = c_ref[...]
        cs_ref[...] = jnp.sum(c0 * c0, axis=1)[None, :]
        n2c_ref[...] = c0 * (-2.0)

    x = x_ref[...]

    input_sq = jnp.sum(x * x, axis=1, keepdims=True)
    mm2 = jnp.dot(x, n2c_ref[...].T, preferred_element_type=jnp.float32)
    d = (input_sq + cs_ref[0, :]) + mm2

    dmin = jnp.min(d, axis=1, keepdims=True)
    col = jax.lax.broadcasted_iota(jnp.int32, d.shape, 1)
    idx = jnp.min(jnp.where(d == dmin, col, N_EMB), axis=1)

    oh = (col == idx[:, None]).astype(jnp.float32)
    q = jnp.dot(oh, c_ref[...], preferred_element_type=jnp.float32)

    q_ref[...] = q
    idx_ref[0, 0, :] = idx

    blk_loss = jnp.sum(dmin)
    ones_row = jnp.ones((1, BLK), jnp.float32)
    blk_counts = jnp.dot(ones_row, oh, preferred_element_type=jnp.float32)

    @pl.when(i == 0)
    def _init():
        counts_ref[...] = blk_counts
        loss_ref[...] = blk_loss.reshape(1, 1)
        perp_ref[...] = jnp.zeros((1, 1), jnp.float32)

    @pl.when(i > 0)
    def _acc():
        counts_ref[...] += blk_counts
        loss_ref[...] += blk_loss.reshape(1, 1)

    @pl.when(i == nsteps - 1)
    def _finish():
        p = counts_ref[0, :] / jnp.float32(total_tokens)
        ent = jnp.sum(p * jnp.log(p + 1e-10))
        perp_ref[...] = jnp.exp(-ent).reshape(1, 1)
        mse = loss_ref[0, 0] / jnp.float32(total_tokens * DIM)
        loss_ref[...] = (mse * COMMITMENT_COST + mse).reshape(1, 1)


def kernel(inputs, codebook):
    input_shape = inputs.shape
    x = inputs.reshape(-1, DIM)
    tokens = x.shape[0]
    grid = tokens // BLK

    q, idx3, vq_loss, perp = pl.pallas_call(
        _vq_block,
        grid=(grid,),
        in_specs=[
            pl.BlockSpec((BLK, DIM), lambda i: (i, 0)),
            pl.BlockSpec((N_EMB, DIM), lambda i: (0, 0)),
        ],
        out_specs=[
            pl.BlockSpec((BLK, DIM), lambda i: (i, 0)),
            pl.BlockSpec((1, 1, BLK), lambda i: (i, 0, 0)),
            pl.BlockSpec((1, 1), lambda i: (0, 0)),
            pl.BlockSpec((1, 1), lambda i: (0, 0)),
        ],
        out_shape=[
            jax.ShapeDtypeStruct((tokens, DIM), jnp.float32),
            jax.ShapeDtypeStruct((grid, 1, BLK), jnp.int32),
            jax.ShapeDtypeStruct((1, 1), jnp.float32),
            jax.ShapeDtypeStruct((1, 1), jnp.float32),
        ],
        scratch_shapes=[
            pltpu.VMEM((1, N_EMB), jnp.float32),
            pltpu.VMEM((1, N_EMB), jnp.float32),
            pltpu.VMEM((N_EMB, DIM), jnp.float32),
        ],
    )(x, codebook)

    quantized = q.reshape(input_shape)
    indices = idx3.reshape(input_shape[:-1])
    return (quantized, vq_loss[0, 0], perp[0, 0], indices)


# speedup vs baseline: 2.2671x; 1.0272x over previous
import jax
import jax.numpy as jnp
from jax.experimental import pallas as pl
from jax.experimental.pallas import tpu as pltpu

N_EMB = 1024
DIM = 64
COMMITMENT_COST = 0.25
BLK = 4096


def _vq_block(x_ref, c_ref, q_ref, idx_ref, loss_ref, perp_ref,
              counts_ref, cs_ref, n2c_ref):
    i = pl.program_id(0)
    nsteps = pl.num_programs(0)
    total_tokens = nsteps * BLK

    @pl.when(i == 0)
    def _prep():
        c0 = c_ref[...]
        cs_ref[...] = jnp.sum(c0 * c0, axis=1, keepdims=True)
        n2c_ref[...] = c0 * (-2.0)

    x = x_ref[...]

    input_sq = jnp.sum(x * x, axis=1, keepdims=True)
    is_row = input_sq.T
    mm2 = jax.lax.dot_general(n2c_ref[...], x, (((1,), (1,)), ((), ())),
                              preferred_element_type=jnp.float32)
    d = (is_row + cs_ref[...]) + mm2

    dmin = jnp.min(d, axis=0, keepdims=True)
    row = jax.lax.broadcasted_iota(jnp.int32, d.shape, 0)
    idx = jnp.min(jnp.where(d == dmin, row, N_EMB), axis=0)

    oh = (row == idx[None, :]).astype(jnp.float32)
    q = jax.lax.dot_general(oh, c_ref[...], (((0,), (0,)), ((), ())),
                            preferred_element_type=jnp.float32)

    q_ref[...] = q
    idx_ref[0, 0, :] = idx

    blk_loss = jnp.sum(dmin)
    ones_col = jnp.ones((BLK, 1), jnp.float32)
    blk_counts = jnp.dot(oh, ones_col,
                         preferred_element_type=jnp.float32)

    @pl.when(i == 0)
    def _init():
        counts_ref[...] = blk_counts
        loss_ref[...] = blk_loss.reshape(1, 1)
        perp_ref[...] = jnp.zeros((1, 1), jnp.float32)

    @pl.when(i > 0)
    def _acc():
        counts_ref[...] += blk_counts
        loss_ref[...] += blk_loss.reshape(1, 1)

    @pl.when(i == nsteps - 1)
    def _finish():
        p = counts_ref[:, 0] / jnp.float32(total_tokens)
        ent = jnp.sum(p * jnp.log(p + 1e-10))
        perp_ref[...] = jnp.exp(-ent).reshape(1, 1)
        mse = loss_ref[0, 0] / jnp.float32(total_tokens * DIM)
        loss_ref[...] = (mse * COMMITMENT_COST + mse).reshape(1, 1)


def kernel(inputs, codebook):
    input_shape = inputs.shape
    x = inputs.reshape(-1, DIM)
    tokens = x.shape[0]
    grid = tokens // BLK

    q, idx3, vq_loss, perp = pl.pallas_call(
        _vq_block,
        grid=(grid,),
        in_specs=[
            pl.BlockSpec((BLK, DIM), lambda i: (i, 0)),
            pl.BlockSpec((N_EMB, DIM), lambda i: (0, 0)),
        ],
        out_specs=[
            pl.BlockSpec((BLK, DIM), lambda i: (i, 0)),
            pl.BlockSpec((1, 1, BLK), lambda i: (i, 0, 0)),
            pl.BlockSpec((1, 1), lambda i: (0, 0)),
            pl.BlockSpec((1, 1), lambda i: (0, 0)),
        ],
        out_shape=[
            jax.ShapeDtypeStruct((tokens, DIM), jnp.float32),
            jax.ShapeDtypeStruct((grid, 1, BLK), jnp.int32),
            jax.ShapeDtypeStruct((1, 1), jnp.float32),
            jax.ShapeDtypeStruct((1, 1), jnp.float32),
        ],
        scratch_shapes=[
            pltpu.VMEM((N_EMB, 1), jnp.float32),
            pltpu.VMEM((N_EMB, 1), jnp.float32),
            pltpu.VMEM((N_EMB, DIM), jnp.float32),
        ],
    )(x, codebook)

    quantized = q.reshape(input_shape)
    indices = idx3.reshape(input_shape[:-1])
    return (quantized, vq_loss[0, 0], perp[0, 0], indices)
